# baseline (speedup 1.0000x reference)
import functools

import jax
import jax.numpy as jnp
from jax import lax
from jax.experimental import pallas as pl
from jax.experimental.pallas import tpu as pltpu
from jax.experimental.pallas import tpu_sc as plsc

_L = 16
_NEG = -3.4e38


@functools.lru_cache(maxsize=None)
def _make_sparsemax(R, N):
  info = plsc.get_sparse_core_info()
  NC, NS = info.num_cores, info.num_subcores
  NW = NC * NS
  assert R % (2 * NW) == 0 and N % _L == 0
  RPW = R // NW
  PAIRS = RPW // 2
  NCHUNK = N // _L

  mesh = plsc.VectorSubcoreMesh(core_axis_name="c", subcore_axis_name="s")

  @functools.partial(
      pl.kernel,
      out_type=jax.ShapeDtypeStruct((R, N), jnp.float32),
      mesh=mesh,
      scratch_types=[
          pltpu.VMEM((N,), jnp.float32),
          pltpu.VMEM((N,), jnp.float32),
          pltpu.VMEM((N,), jnp.float32),
          pltpu.VMEM((N,), jnp.float32),
          pltpu.VMEM((N + _L,), jnp.int32),
          pltpu.SemaphoreType.DMA,
          pltpu.SemaphoreType.DMA,
          pltpu.SemaphoreType.DMA,
          pltpu.SemaphoreType.DMA,
      ],
      compiler_params=pltpu.CompilerParams(needs_layout_passes=False),
  )
  def ker(x_hbm, out_hbm, zbuf0, zbuf1, obuf0, obuf1, idxs,
          sin0, sin1, sout0, sout1):
    cid = lax.axis_index("c")
    sid = lax.axis_index("s")
    base = (sid * NC + cid) * RPW

    zf = jnp.zeros((_L,), jnp.float32)
    zi = jnp.zeros((_L,), jnp.int32)
    iota = lax.iota(jnp.int32, _L)
    negv = jnp.full((_L,), _NEG, jnp.float32)

    def scal(v):
      return lax.squeeze(lax.slice(v, (0,), (1,)), (0,))

    def process_row(zbuf, obuf):
      U1 = 8
      def mx(i, accs):
        res = []
        for u in range(U1):
          b = (i * U1 + u) * _L
          obuf[pl.ds(b, _L)] = zf
          res.append(jnp.maximum(accs[u], zbuf[pl.ds(b, _L)]))
        return tuple(res)
      accs = lax.fori_loop(0, NCHUNK // U1, mx, (negv,) * U1)
      acc = functools.reduce(jnp.maximum, accs)
      thr = jnp.full((_L,), jnp.max(acc), jnp.float32) - 1.0

      def flt(i, carry):
        off, iv = carry
        v = zbuf[pl.ds(i * _L, _L)]
        m = v > thr
        plsc.store_compressed(idxs.at[pl.ds(off, _L)], iv, mask=m)
        off = off + scal(plsc.all_reduce_population_count(m))
        return (off, iv + _L)
      c0, _ = lax.fori_loop(0, NCHUNK, flt,
                            (jnp.asarray(0, jnp.int32), iota), unroll=8)
      nch = (c0 + (_L - 1)) >> 4

      def gather_cand(i, lm):
        ix = idxs[pl.ds(i * _L, _L)]
        return ix, plsc.load_gather(zbuf, [ix], mask=lm)

      c0v = jnp.full((_L,), c0, jnp.int32)

      def w_cond(st):
        return jnp.logical_not(st[2])

      def w_body(st):
        tau, c_prev, _ = st
        def ch(i, carry):
          sa, ca = carry
          lm = (i * _L + iota) < c0v
          _, v = gather_cand(i, lm)
          m = jnp.logical_and(v > tau, lm)
          return (sa + jnp.where(m, v, zf), ca + m.astype(jnp.int32))
        sa, ca = lax.fori_loop(0, nch, ch, (zf, zi))
        c = jnp.sum(ca)
        tau_new = (jnp.full((_L,), jnp.sum(sa), jnp.float32) - 1.0) / (
            jnp.full((_L,), c, jnp.int32).astype(jnp.float32))
        return (tau_new, c, c == c_prev)

      tau, _, _ = lax.while_loop(
          w_cond, w_body,
          (thr, jnp.asarray(-1, jnp.int32), jnp.asarray(False)))

      def outp(i, carry):
        lm = (i * _L + iota) < c0v
        ix, v = gather_cand(i, lm)
        m = jnp.logical_and(v > tau, lm)
        plsc.store_scatter(obuf, [ix], v - tau, mask=m)
        return carry
      lax.fori_loop(0, nch, outp, 0)

    def row_pair(j, carry):
      r0 = base + 2 * j
      r1 = r0 + 1
      pltpu.make_async_copy(x_hbm.at[r0], zbuf0, sin0).wait()
      pltpu.make_async_copy(x_hbm.at[r1], zbuf1, sin1).start()

      @pl.when(j > 0)
      def _():
        pltpu.make_async_copy(obuf0, out_hbm.at[r0 - 2], sout0).wait()
      process_row(zbuf0, obuf0)
      pltpu.make_async_copy(obuf0, out_hbm.at[r0], sout0).start()

      pltpu.make_async_copy(x_hbm.at[r1], zbuf1, sin1).wait()

      @pl.when(j < PAIRS - 1)
      def _():
        pltpu.make_async_copy(x_hbm.at[r1 + 1], zbuf0, sin0).start()

      @pl.when(j > 0)
      def _():
        pltpu.make_async_copy(obuf1, out_hbm.at[r1 - 2], sout1).wait()
      process_row(zbuf1, obuf1)
      pltpu.make_async_copy(obuf1, out_hbm.at[r1], sout1).start()
      return carry

    pltpu.make_async_copy(x_hbm.at[base], zbuf0, sin0).start()
    lax.fori_loop(0, PAIRS, row_pair, 0)
    pltpu.make_async_copy(obuf0, out_hbm.at[base + RPW - 2], sout0).wait()
    pltpu.make_async_copy(obuf1, out_hbm.at[base + RPW - 1], sout1).wait()

  return ker


@jax.jit
def _sparsemax2d(x):
  return _make_sparsemax(*x.shape)(x)


def kernel(inputs, mask):
  del mask
  b, q, n = inputs.shape
  return _sparsemax2d(inputs.reshape(b * q, n)).reshape(b, q, n)


# speedup vs baseline: 1.2151x; 1.2151x over previous
import functools

import jax
import jax.numpy as jnp
from jax import lax
from jax.experimental import pallas as pl
from jax.experimental.pallas import tpu as pltpu
from jax.experimental.pallas import tpu_sc as plsc

_L = 16
_NEG = -3.4e38


@functools.lru_cache(maxsize=None)
def _make_sparsemax(R, N):
  info = plsc.get_sparse_core_info()
  NC, NS = info.num_cores, info.num_subcores
  NW = NC * NS
  assert R % (2 * NW) == 0 and N % _L == 0
  RPW = R // NW
  PAIRS = RPW // 2
  NCHUNK = N // _L

  mesh = plsc.VectorSubcoreMesh(core_axis_name="c", subcore_axis_name="s")

  @functools.partial(
      pl.kernel,
      out_type=jax.ShapeDtypeStruct((R, N), jnp.float32),
      mesh=mesh,
      scratch_types=[
          pltpu.VMEM((N,), jnp.float32),
          pltpu.VMEM((N,), jnp.float32),
          pltpu.VMEM((N,), jnp.float32),
          pltpu.VMEM((N,), jnp.float32),
          pltpu.VMEM((N + _L,), jnp.int32),
          pltpu.SemaphoreType.DMA,
          pltpu.SemaphoreType.DMA,
          pltpu.SemaphoreType.DMA,
          pltpu.SemaphoreType.DMA,
      ],
      compiler_params=pltpu.CompilerParams(needs_layout_passes=False),
  )
  def ker(x_hbm, out_hbm, zbuf0, zbuf1, obuf0, obuf1, idxs,
          sin0, sin1, sout0, sout1):
    cid = lax.axis_index("c")
    sid = lax.axis_index("s")
    base = (sid * NC + cid) * RPW

    zf = jnp.zeros((_L,), jnp.float32)
    zi = jnp.zeros((_L,), jnp.int32)
    iota = lax.iota(jnp.int32, _L)
    negv = jnp.full((_L,), _NEG, jnp.float32)

    def scal(v):
      return lax.squeeze(lax.slice(v, (0,), (1,)), (0,))

    def process_row(zbuf, obuf):
      U1 = 8
      def mx(i, accs):
        res = []
        for u in range(U1):
          b = (i * U1 + u) * _L
          obuf[pl.ds(b, _L)] = zf
          res.append(jnp.maximum(accs[u], zbuf[pl.ds(b, _L)]))
        return tuple(res)
      accs = lax.fori_loop(0, NCHUNK // U1, mx, (negv,) * U1)
      acc = functools.reduce(jnp.maximum, accs)
      thr = jnp.full((_L,), jnp.max(acc), jnp.float32) - 1.0

      def flt(i, carry):
        cnt, iv = carry
        v = zbuf[pl.ds(i * _L, _L)]
        m = v > thr
        plsc.store_scatter(idxs, [cnt * _L + iota], iv, mask=m)
        return (cnt + m.astype(jnp.int32), iv + _L)
      cnt, _ = lax.fori_loop(0, NCHUNK, flt, (zi, iota), unroll=8)
      maxc = jnp.max(cnt)

      def gather_cand(i, lm):
        ix = idxs[pl.ds(i * _L, _L)]
        return ix, plsc.load_gather(zbuf, [ix], mask=lm)

      def w_cond(st):
        return jnp.logical_not(st[2])

      def w_body(st):
        tau, c_prev, _ = st
        def ch(i, carry):
          sa, ca = carry
          lm = i < cnt
          _, v = gather_cand(i, lm)
          m = jnp.logical_and(v > tau, lm)
          return (sa + jnp.where(m, v, zf), ca + m.astype(jnp.int32))
        sa, ca = lax.fori_loop(0, maxc, ch, (zf, zi))
        c = jnp.sum(ca)
        tau_new = (jnp.full((_L,), jnp.sum(sa), jnp.float32) - 1.0) / (
            jnp.full((_L,), c, jnp.int32).astype(jnp.float32))
        return (tau_new, c, c == c_prev)

      tau, _, _ = lax.while_loop(
          w_cond, w_body,
          (thr, jnp.asarray(-1, jnp.int32), jnp.asarray(False)))

      def outp(i, carry):
        lm = i < cnt
        ix, v = gather_cand(i, lm)
        m = jnp.logical_and(v > tau, lm)
        plsc.store_scatter(obuf, [ix], v - tau, mask=m)
        return carry
      lax.fori_loop(0, maxc, outp, 0)

    def row_pair(j, carry):
      r0 = base + 2 * j
      r1 = r0 + 1
      pltpu.make_async_copy(x_hbm.at[r0], zbuf0, sin0).wait()
      pltpu.make_async_copy(x_hbm.at[r1], zbuf1, sin1).start()

      @pl.when(j > 0)
      def _():
        pltpu.make_async_copy(obuf0, out_hbm.at[r0 - 2], sout0).wait()
      process_row(zbuf0, obuf0)
      pltpu.make_async_copy(obuf0, out_hbm.at[r0], sout0).start()

      pltpu.make_async_copy(x_hbm.at[r1], zbuf1, sin1).wait()

      @pl.when(j < PAIRS - 1)
      def _():
        pltpu.make_async_copy(x_hbm.at[r1 + 1], zbuf0, sin0).start()

      @pl.when(j > 0)
      def _():
        pltpu.make_async_copy(obuf1, out_hbm.at[r1 - 2], sout1).wait()
      process_row(zbuf1, obuf1)
      pltpu.make_async_copy(obuf1, out_hbm.at[r1], sout1).start()
      return carry

    pltpu.make_async_copy(x_hbm.at[base], zbuf0, sin0).start()
    lax.fori_loop(0, PAIRS, row_pair, 0)
    pltpu.make_async_copy(obuf0, out_hbm.at[base + RPW - 2], sout0).wait()
    pltpu.make_async_copy(obuf1, out_hbm.at[base + RPW - 1], sout1).wait()

  return ker


@jax.jit
def _sparsemax2d(x):
  return _make_sparsemax(*x.shape)(x)


def kernel(inputs, mask):
  del mask
  b, q, n = inputs.shape
  return _sparsemax2d(inputs.reshape(b * q, n)).reshape(b, q, n)


# speedup vs baseline: 1.9928x; 1.6400x over previous
import functools

import jax
import jax.numpy as jnp
from jax import lax
from jax.experimental import pallas as pl
from jax.experimental.pallas import tpu as pltpu
from jax.experimental.pallas import tpu_sc as plsc

_L = 16
_NEG = -3.4e38


@functools.lru_cache(maxsize=None)
def _make_sparsemax(R, N):
  info = plsc.get_sparse_core_info()
  NC, NS = info.num_cores, info.num_subcores
  NW = NC * NS
  assert R % (2 * NW) == 0 and N % _L == 0
  RPW = R // NW
  PAIRS = RPW // 2
  NCHUNK = N // _L

  mesh = plsc.VectorSubcoreMesh(core_axis_name="c", subcore_axis_name="s")

  @functools.partial(
      pl.kernel,
      out_type=jax.ShapeDtypeStruct((R, N), jnp.float32),
      mesh=mesh,
      scratch_types=[
          pltpu.VMEM((N,), jnp.float32),
          pltpu.VMEM((N,), jnp.float32),
          pltpu.VMEM((N,), jnp.float32),
          pltpu.VMEM((N,), jnp.float32),
          pltpu.VMEM((N + _L,), jnp.int32),
          pltpu.SemaphoreType.DMA,
          pltpu.SemaphoreType.DMA,
          pltpu.SemaphoreType.DMA,
          pltpu.SemaphoreType.DMA,
      ],
      compiler_params=pltpu.CompilerParams(needs_layout_passes=False),
  )
  def ker(x_hbm, out_hbm, zbuf0, zbuf1, obuf0, obuf1, idxs,
          sin0, sin1, sout0, sout1):
    cid = lax.axis_index("c")
    sid = lax.axis_index("s")
    base = (sid * NC + cid) * RPW

    zf = jnp.zeros((_L,), jnp.float32)
    zi = jnp.zeros((_L,), jnp.int32)
    iota = lax.iota(jnp.int32, _L)
    negv = jnp.full((_L,), _NEG, jnp.float32)

    def scal(v):
      return lax.squeeze(lax.slice(v, (0,), (1,)), (0,))

    def process_row(zbuf, obuf):
      U1 = 8
      def mx(i, accs):
        res = []
        for u in range(U1):
          b = (i * U1 + u) * _L
          obuf[pl.ds(b, _L)] = zf
          res.append(jnp.maximum(accs[u], zbuf[pl.ds(b, _L)]))
        return tuple(res)
      accs = lax.fori_loop(0, NCHUNK // U1, mx, (negv,) * U1)
      acc = functools.reduce(jnp.maximum, accs)
      thr = jnp.full((_L,), jnp.max(acc), jnp.float32) - 1.0

      G = 8
      def flt(g, carry):
        cnt, iv = carry
        ms = []
        m_or = None
        for u in range(G):
          v = zbuf[pl.ds((g * G + u) * _L, _L)]
          m = v > thr
          ms.append(m)
          m_or = m if m_or is None else jnp.logical_or(m_or, m)
        cnts = [cnt]
        for u in range(G):
          cnts.append(cnts[-1] + ms[u].astype(jnp.int32))
        pc = plsc.all_reduce_population_count(m_or)

        @pl.when(scal(pc) > 0)
        def _():
          for u in range(G):
            plsc.store_scatter(idxs, [cnts[u] * _L + iota],
                               iv + u * _L, mask=ms[u])
        return (cnts[-1], iv + G * _L)
      cnt, _ = lax.fori_loop(0, NCHUNK // G, flt, (zi, iota))
      maxc = jnp.max(cnt)

      def gather_cand(i, lm):
        ix = idxs[pl.ds(i * _L, _L)]
        return ix, plsc.load_gather(zbuf, [ix], mask=lm)

      def w_cond(st):
        return jnp.logical_not(st[2])

      def w_body(st):
        tau, c_prev, _ = st
        def ch(i, carry):
          sa, ca = carry
          lm = i < cnt
          _, v = gather_cand(i, lm)
          m = jnp.logical_and(v > tau, lm)
          return (sa + jnp.where(m, v, zf), ca + m.astype(jnp.int32))
        sa, ca = lax.fori_loop(0, maxc, ch, (zf, zi))
        c = jnp.sum(ca)
        tau_new = (jnp.full((_L,), jnp.sum(sa), jnp.float32) - 1.0) / (
            jnp.full((_L,), c, jnp.int32).astype(jnp.float32))
        return (tau_new, c, c == c_prev)

      tau, _, _ = lax.while_loop(
          w_cond, w_body,
          (thr, jnp.asarray(-1, jnp.int32), jnp.asarray(False)))

      def outp(i, carry):
        lm = i < cnt
        ix, v = gather_cand(i, lm)
        m = jnp.logical_and(v > tau, lm)
        plsc.store_scatter(obuf, [ix], v - tau, mask=m)
        return carry
      lax.fori_loop(0, maxc, outp, 0)

    def row_pair(j, carry):
      r0 = base + 2 * j
      r1 = r0 + 1
      pltpu.make_async_copy(x_hbm.at[r0], zbuf0, sin0).wait()
      pltpu.make_async_copy(x_hbm.at[r1], zbuf1, sin1).start()

      @pl.when(j > 0)
      def _():
        pltpu.make_async_copy(obuf0, out_hbm.at[r0 - 2], sout0).wait()
      process_row(zbuf0, obuf0)
      pltpu.make_async_copy(obuf0, out_hbm.at[r0], sout0).start()

      pltpu.make_async_copy(x_hbm.at[r1], zbuf1, sin1).wait()

      @pl.when(j < PAIRS - 1)
      def _():
        pltpu.make_async_copy(x_hbm.at[r1 + 1], zbuf0, sin0).start()

      @pl.when(j > 0)
      def _():
        pltpu.make_async_copy(obuf1, out_hbm.at[r1 - 2], sout1).wait()
      process_row(zbuf1, obuf1)
      pltpu.make_async_copy(obuf1, out_hbm.at[r1], sout1).start()
      return carry

    pltpu.make_async_copy(x_hbm.at[base], zbuf0, sin0).start()
    lax.fori_loop(0, PAIRS, row_pair, 0)
    pltpu.make_async_copy(obuf0, out_hbm.at[base + RPW - 2], sout0).wait()
    pltpu.make_async_copy(obuf1, out_hbm.at[base + RPW - 1], sout1).wait()

  return ker


@jax.jit
def _sparsemax2d(x):
  return _make_sparsemax(*x.shape)(x)


def kernel(inputs, mask):
  del mask
  b, q, n = inputs.shape
  return _sparsemax2d(inputs.reshape(b * q, n)).reshape(b, q, n)
